# Initial kernel scaffold; baseline (speedup 1.0000x reference)
import jax
import jax.numpy as jnp
from jax.experimental import pallas as pl


def kernel(x, edge_index, W1, b1, W2, b2, W_out, b_out):
    raise NotImplementedError("write your pallas kernel here")


# speedup vs baseline: 20.2018x; 20.2018x over previous
import functools

import jax
import jax.numpy as jnp
from jax import lax
from jax.experimental import pallas as pl
from jax.experimental.pallas import tpu as pltpu
from jax.experimental.pallas import tpu_sc as plsc

N = 10000
E = 320000
D = 128

NC = 2
NS = 16
NW = NC * NS
EPT = E // NW
K = 100
NCH = EPT // K
NPAIR = NCH // 2
NPAD = 10240
RPT = NPAD // NS
DEG_W = 16

_mesh = plsc.VectorSubcoreMesh(core_axis_name="c", subcore_axis_name="s")


@functools.partial(
    pl.kernel,
    out_type=jax.ShapeDtypeStruct((NC, NPAD, DEG_W), jnp.float32),
    mesh=_mesh,
    scratch_types=[
        pltpu.VMEM((NCH, K), jnp.int32),
        pltpu.VMEM((K, DEG_W), jnp.float32),
        pltpu.SemaphoreType.DMA,
        pltpu.VMEM_SHARED((NPAD, DEG_W), jnp.float32),
    ],
)
def _deg_kernel(dst_hbm, z_hbm, out_hbm, dst_v, ones_v, sem, deg_sh):
    c_id = lax.axis_index("c")
    s_id = lax.axis_index("s")
    wid = c_id * NS + s_id

    pltpu.sync_copy(dst_hbm.at[wid], dst_v)

    @pl.when(s_id == 0)
    def _():
        pltpu.sync_copy(z_hbm, deg_sh)

    def fill(i, carry):
        ones_v[i] = jnp.ones((DEG_W,), jnp.float32)
        return carry

    lax.fori_loop(0, K, fill, 0)
    plsc.subcore_barrier()

    def step(i, carry):
        pltpu.async_copy(ones_v, deg_sh.at[dst_v.at[i]], sem, add=True).wait()
        return carry

    lax.fori_loop(0, NCH, step, 0)
    plsc.subcore_barrier()

    pltpu.sync_copy(
        deg_sh.at[pl.ds(s_id * RPT, RPT)],
        out_hbm.at[c_id, pl.ds(s_id * RPT, RPT)],
    )


@functools.partial(
    pl.kernel,
    out_type=jax.ShapeDtypeStruct((NC, NPAD, D), jnp.float32),
    mesh=_mesh,
    scratch_types=[
        pltpu.VMEM((NCH, K), jnp.int32),
        pltpu.VMEM((NCH, K), jnp.int32),
        pltpu.VMEM((K, D), jnp.float32),
        pltpu.SemaphoreType.DMA,
        pltpu.SemaphoreType.DMA,
        pltpu.VMEM_SHARED((NPAD, D), jnp.float32),
    ],
)
def _agg_kernel(h_hbm, src_hbm, dst_hbm, z_hbm, out_hbm,
                src_v, dst_v, rows0, gsem, ssem, agg_sh):
    c_id = lax.axis_index("c")
    s_id = lax.axis_index("s")
    wid = c_id * NS + s_id

    pltpu.sync_copy(src_hbm.at[wid], src_v)
    pltpu.sync_copy(dst_hbm.at[wid], dst_v)

    @pl.when(s_id == 0)
    def _():
        pltpu.sync_copy(z_hbm, agg_sh)

    plsc.subcore_barrier()

    def step(c, carry):
        pltpu.async_copy(h_hbm.at[src_v.at[c]], rows0, gsem).wait()
        pltpu.async_copy(rows0, agg_sh.at[dst_v.at[c]], ssem, add=True).wait()
        return carry

    lax.fori_loop(0, NCH, step, 0)
    plsc.subcore_barrier()

    pltpu.sync_copy(
        agg_sh.at[pl.ds(s_id * RPT, RPT)],
        out_hbm.at[c_id, pl.ds(s_id * RPT, RPT)],
    )


BN = 1000
NB = N // BN


def _dinv_of(deg_ref):
    deg = deg_ref[0][:, 0:1] + deg_ref[1][:, 0:1] + 1.0
    return lax.rsqrt(deg)


def _tc1_body(deg_ref, x_ref, w_ref, h_ref, hp_ref):
    dinv = _dinv_of(deg_ref)
    h = jnp.dot(x_ref[...], w_ref[...], preferred_element_type=jnp.float32)
    h_ref[...] = h
    hp_ref[...] = h * dinv


def _tc2_body(p_ref, h1_ref, deg_ref, b_ref, w_ref, h2_ref, h2p_ref):
    dinv = _dinv_of(deg_ref)
    a = p_ref[0] + p_ref[1]
    y = a * dinv + h1_ref[...] * (dinv * dinv) + b_ref[...]
    y = jnp.maximum(y, 0.0)
    h2 = jnp.dot(y, w_ref[...], preferred_element_type=jnp.float32)
    h2_ref[...] = h2
    h2p_ref[...] = h2 * dinv


def _tc3_body(p_ref, h2_ref, deg_ref, b_ref, wo_ref, bo_ref, out_ref):
    dinv = _dinv_of(deg_ref)
    a = p_ref[0] + p_ref[1]
    y = a * dinv + h2_ref[...] * (dinv * dinv) + b_ref[...]
    out_ref[...] = (
        jnp.dot(y, wo_ref[...], preferred_element_type=jnp.float32)
        + bo_ref[...]
    )


_deg_spec = pl.BlockSpec((2, BN, DEG_W), lambda i: (0, i, 0))
_row_spec = pl.BlockSpec((BN, D), lambda i: (i, 0))
_p_spec = pl.BlockSpec((2, BN, D), lambda i: (0, i, 0))
_w_spec = pl.BlockSpec((D, D), lambda i: (0, 0))
_b_spec = pl.BlockSpec((1, D), lambda i: (0, 0))

_tc1 = pl.pallas_call(
    _tc1_body,
    grid=(NB,),
    in_specs=[_deg_spec, _row_spec, _w_spec],
    out_specs=[_row_spec, _row_spec],
    out_shape=[
        jax.ShapeDtypeStruct((N, D), jnp.float32),
        jax.ShapeDtypeStruct((N, D), jnp.float32),
    ],
)

_tc2 = pl.pallas_call(
    _tc2_body,
    grid=(NB,),
    in_specs=[_p_spec, _row_spec, _deg_spec, _b_spec, _w_spec],
    out_specs=[_row_spec, _row_spec],
    out_shape=[
        jax.ShapeDtypeStruct((N, D), jnp.float32),
        jax.ShapeDtypeStruct((N, D), jnp.float32),
    ],
)

_tc3 = pl.pallas_call(
    _tc3_body,
    grid=(NB,),
    in_specs=[
        _p_spec, _row_spec, _deg_spec, _b_spec,
        pl.BlockSpec((D, 1), lambda i: (0, 0)),
        pl.BlockSpec((1, 1), lambda i: (0, 0)),
    ],
    out_specs=pl.BlockSpec((BN, 1), lambda i: (i, 0)),
    out_shape=jax.ShapeDtypeStruct((N, 1), jnp.float32),
)


def kernel(x, edge_index, W1, b1, W2, b2, W_out, b_out):
    src = edge_index[0].reshape(NW, NCH, K)
    dst = edge_index[1].reshape(NW, NCH, K)
    z128 = jnp.zeros((NPAD, D), jnp.float32)
    z16 = jnp.zeros((NPAD, DEG_W), jnp.float32)

    degp = _deg_kernel(dst, z16)
    h1, h1p = _tc1(degp, x, W1)
    p1 = _agg_kernel(h1p, src, dst, z128)
    h2, h2p = _tc2(p1, h1, degp, b1.reshape(1, D), W2)
    p2 = _agg_kernel(h2p, src, dst, z128)
    out = _tc3(p2, h2, degp, b2.reshape(1, D), W_out, b_out.reshape(1, 1))
    return out


# speedup vs baseline: 24.2163x; 1.1987x over previous
import functools

import jax
import jax.numpy as jnp
from jax import lax
from jax.experimental import pallas as pl
from jax.experimental.pallas import tpu as pltpu
from jax.experimental.pallas import tpu_sc as plsc

N = 10000
E = 320000
D = 128

NC = 2
NS = 16
NW = NC * NS
EPT = E // NW
K = 100
NCH = EPT // K
NPAIR = NCH // 2
NPAD = 10240
RPT = NPAD // NS
DEG_W = 16

_mesh = plsc.VectorSubcoreMesh(core_axis_name="c", subcore_axis_name="s")


@functools.partial(
    pl.kernel,
    out_type=jax.ShapeDtypeStruct((NC, NPAD, DEG_W), jnp.float32),
    mesh=_mesh,
    scratch_types=[
        pltpu.VMEM((NCH, K), jnp.int32),
        pltpu.VMEM((K, DEG_W), jnp.float32),
        pltpu.SemaphoreType.DMA,
        pltpu.VMEM_SHARED((NPAD, DEG_W), jnp.float32),
    ],
)
def _deg_kernel(dst_hbm, z_hbm, out_hbm, dst_v, ones_v, sem, deg_sh):
    c_id = lax.axis_index("c")
    s_id = lax.axis_index("s")
    wid = c_id * NS + s_id

    pltpu.sync_copy(dst_hbm.at[wid], dst_v)

    @pl.when(s_id == 0)
    def _():
        pltpu.sync_copy(z_hbm, deg_sh)

    def fill(i, carry):
        ones_v[i] = jnp.ones((DEG_W,), jnp.float32)
        return carry

    lax.fori_loop(0, K, fill, 0)
    plsc.subcore_barrier()

    def step(i, carry):
        pltpu.async_copy(ones_v, deg_sh.at[dst_v.at[i]], sem, add=True).wait()
        return carry

    lax.fori_loop(0, NCH, step, 0)
    plsc.subcore_barrier()

    pltpu.sync_copy(
        deg_sh.at[pl.ds(s_id * RPT, RPT)],
        out_hbm.at[c_id, pl.ds(s_id * RPT, RPT)],
    )


NBLK = 5
BC = NCH // NBLK


@functools.partial(
    pl.kernel,
    out_type=jax.ShapeDtypeStruct((NC, NPAD, D), jnp.float32),
    mesh=_mesh,
    scratch_types=[
        pltpu.VMEM((BC, K), jnp.int32),
        pltpu.VMEM((BC, K), jnp.int32),
        pltpu.VMEM((BC, K), jnp.int32),
        pltpu.VMEM((BC, K), jnp.int32),
        pltpu.VMEM((K, D), jnp.float32),
        pltpu.VMEM((K, D), jnp.float32),
        pltpu.SemaphoreType.DMA,
        pltpu.SemaphoreType.DMA,
        pltpu.SemaphoreType.DMA,
        pltpu.SemaphoreType.DMA,
        pltpu.SemaphoreType.DMA,
        pltpu.VMEM_SHARED((NPAD, D), jnp.float32),
    ],
)
def _agg_kernel(h_hbm, src_hbm, dst_hbm, z_hbm, out_hbm,
                src_b0, src_b1, dst_b0, dst_b1, rows0, rows1,
                gsem0, gsem1, ssem0, ssem1, isem, agg_sh):
    src_bufs = (src_b0, src_b1)
    dst_bufs = (dst_b0, dst_b1)
    c_id = lax.axis_index("c")
    s_id = lax.axis_index("s")
    wid = c_id * NS + s_id

    pltpu.sync_copy(src_hbm.at[wid, 0], src_b0)
    pltpu.sync_copy(dst_hbm.at[wid, 0], dst_b0)

    @pl.when(s_id == 0)
    def _():
        pltpu.sync_copy(z_hbm, agg_sh)

    plsc.subcore_barrier()

    pltpu.async_copy(h_hbm.at[src_b0.at[0]], rows0, gsem0)
    pltpu.async_copy(h_hbm.at[src_b0.at[1]], rows1, gsem1)

    for b in range(NBLK):
        sv = src_bufs[b % 2]
        dv = dst_bufs[b % 2]
        svn = src_bufs[(b + 1) % 2]
        dvn = dst_bufs[(b + 1) % 2]
        if b + 1 < NBLK:
            pltpu.async_copy(src_hbm.at[wid, b + 1], svn, isem)
            pltpu.async_copy(dst_hbm.at[wid, b + 1], dvn, isem)

        def pair(i, carry):
            c0 = 2 * i
            c1 = c0 + 1
            pltpu.make_async_copy(h_hbm.at[sv.at[c0]], rows0, gsem0).wait()
            pltpu.async_copy(rows0, agg_sh.at[dv.at[c0]], ssem0, add=True)
            pltpu.make_async_copy(h_hbm.at[sv.at[c1]], rows1, gsem1).wait()
            pltpu.async_copy(rows1, agg_sh.at[dv.at[c1]], ssem1, add=True)

            @pl.when(i + 1 < BC // 2)
            def _():
                pltpu.make_async_copy(rows0, agg_sh.at[dv.at[c0]], ssem0).wait()
                pltpu.async_copy(h_hbm.at[sv.at[c0 + 2]], rows0, gsem0)
                pltpu.make_async_copy(rows1, agg_sh.at[dv.at[c1]], ssem1).wait()
                pltpu.async_copy(h_hbm.at[sv.at[c1 + 2]], rows1, gsem1)

            return carry

        lax.fori_loop(0, BC // 2, pair, 0)

        if b + 1 < NBLK:
            pltpu.make_async_copy(src_hbm.at[wid, b + 1], svn, isem).wait()
            pltpu.make_async_copy(dst_hbm.at[wid, b + 1], dvn, isem).wait()
            pltpu.make_async_copy(rows0, agg_sh.at[dv.at[BC - 2]], ssem0).wait()
            pltpu.async_copy(h_hbm.at[svn.at[0]], rows0, gsem0)
            pltpu.make_async_copy(rows1, agg_sh.at[dv.at[BC - 1]], ssem1).wait()
            pltpu.async_copy(h_hbm.at[svn.at[1]], rows1, gsem1)
        else:
            pltpu.make_async_copy(rows0, agg_sh.at[dv.at[BC - 2]], ssem0).wait()
            pltpu.make_async_copy(rows1, agg_sh.at[dv.at[BC - 1]], ssem1).wait()

    plsc.subcore_barrier()

    pltpu.sync_copy(
        agg_sh.at[pl.ds(s_id * RPT, RPT)],
        out_hbm.at[c_id, pl.ds(s_id * RPT, RPT)],
    )


BN = 1000
NB = N // BN


def _dinv_of(deg_ref):
    deg = deg_ref[0][:, 0:1] + deg_ref[1][:, 0:1] + 1.0
    return lax.rsqrt(deg)


def _tc1_body(deg_ref, x_ref, w_ref, h_ref, hp_ref):
    dinv = _dinv_of(deg_ref)
    h = jnp.dot(x_ref[...], w_ref[...], preferred_element_type=jnp.float32)
    h_ref[...] = h
    hp_ref[...] = h * dinv


def _tc2_body(p_ref, h1_ref, deg_ref, b_ref, w_ref, h2_ref, h2p_ref):
    dinv = _dinv_of(deg_ref)
    a = p_ref[0] + p_ref[1]
    y = a * dinv + h1_ref[...] * (dinv * dinv) + b_ref[...]
    y = jnp.maximum(y, 0.0)
    h2 = jnp.dot(y, w_ref[...], preferred_element_type=jnp.float32)
    h2_ref[...] = h2
    h2p_ref[...] = h2 * dinv


def _tc3_body(p_ref, h2_ref, deg_ref, b_ref, wo_ref, bo_ref, out_ref):
    dinv = _dinv_of(deg_ref)
    a = p_ref[0] + p_ref[1]
    y = a * dinv + h2_ref[...] * (dinv * dinv) + b_ref[...]
    out_ref[...] = (
        jnp.dot(y, wo_ref[...], preferred_element_type=jnp.float32)
        + bo_ref[...]
    )


_deg_spec = pl.BlockSpec((2, BN, DEG_W), lambda i: (0, i, 0))
_row_spec = pl.BlockSpec((BN, D), lambda i: (i, 0))
_p_spec = pl.BlockSpec((2, BN, D), lambda i: (0, i, 0))
_w_spec = pl.BlockSpec((D, D), lambda i: (0, 0))
_b_spec = pl.BlockSpec((1, D), lambda i: (0, 0))

_tc1 = pl.pallas_call(
    _tc1_body,
    grid=(NB,),
    in_specs=[_deg_spec, _row_spec, _w_spec],
    out_specs=[_row_spec, _row_spec],
    out_shape=[
        jax.ShapeDtypeStruct((N, D), jnp.float32),
        jax.ShapeDtypeStruct((N, D), jnp.float32),
    ],
)

_tc2 = pl.pallas_call(
    _tc2_body,
    grid=(NB,),
    in_specs=[_p_spec, _row_spec, _deg_spec, _b_spec, _w_spec],
    out_specs=[_row_spec, _row_spec],
    out_shape=[
        jax.ShapeDtypeStruct((N, D), jnp.float32),
        jax.ShapeDtypeStruct((N, D), jnp.float32),
    ],
)

_tc3 = pl.pallas_call(
    _tc3_body,
    grid=(NB,),
    in_specs=[
        _p_spec, _row_spec, _deg_spec, _b_spec,
        pl.BlockSpec((D, 1), lambda i: (0, 0)),
        pl.BlockSpec((1, 1), lambda i: (0, 0)),
    ],
    out_specs=pl.BlockSpec((BN, 1), lambda i: (i, 0)),
    out_shape=jax.ShapeDtypeStruct((N, 1), jnp.float32),
)


def kernel(x, edge_index, W1, b1, W2, b2, W_out, b_out):
    src = edge_index[0].reshape(NW, NBLK, BC, K)
    dst = edge_index[1].reshape(NW, NBLK, BC, K)
    dst_flat = edge_index[1].reshape(NW, NCH, K)
    z128 = jnp.zeros((NPAD, D), jnp.float32)
    z16 = jnp.zeros((NPAD, DEG_W), jnp.float32)

    degp = _deg_kernel(dst_flat, z16)
    h1, h1p = _tc1(degp, x, W1)
    p1 = _agg_kernel(h1p, src, dst, z128)
    h2, h2p = _tc2(p1, h1, degp, b1.reshape(1, D), W2)
    p2 = _agg_kernel(h2p, src, dst, z128)
    out = _tc3(p2, h2, degp, b2.reshape(1, D), W_out, b_out.reshape(1, 1))
    return out
